# Initial kernel scaffold; baseline (speedup 1.0000x reference)
import jax
import jax.numpy as jnp
from jax.experimental import pallas as pl


def kernel(positions, edge_index, batch, local_or_ghost, cell, ptr, pair_weight):
    raise NotImplementedError("write your pallas kernel here")


# speedup vs baseline: 425.1931x; 425.1931x over previous
import functools

import jax
import jax.numpy as jnp
from jax import lax
from jax.experimental import pallas as pl
from jax.experimental.pallas import tpu as pltpu
from jax.experimental.pallas import tpu_sc as plsc
from jax._src.pallas.mosaic import helpers as plm_helpers

N_ATOMS = 100000
N_EDGES = 3200000
NC = 2
NS = 16
NW = NC * NS
NA = 100096
CHUNK = 512
EW = 100352
EP = EW * NW
NCH = EW // CHUNK
AS = NA // NS

_PCOL = 4


def _sc_accumulate(send, recv, pos8, w16, z16):
    mesh = plsc.VectorSubcoreMesh(core_axis_name="c", subcore_axis_name="s")

    @functools.partial(
        pl.kernel,
        out_type=jax.ShapeDtypeStruct((NC, NA, 16), jnp.float32),
        mesh=mesh,
        compiler_params=pltpu.CompilerParams(
            use_tc_tiling_on_sc=False, needs_layout_passes=False),
        scratch_types=(
            pltpu.VMEM_SHARED((NA, 16), jnp.float32),
            pltpu.VMEM((CHUNK,), jnp.int32),
            pltpu.VMEM((CHUNK,), jnp.int32),
            pltpu.VMEM((CHUNK, 8), jnp.float32),
            pltpu.VMEM((CHUNK, 8), jnp.float32),
            pltpu.VMEM((CHUNK, 16), jnp.float32),
            pltpu.VMEM((CHUNK, 16), jnp.float32),
            pltpu.VMEM((3, 16), jnp.float32),
        ),
    )
    def kern(send_h, recv_h, pos_h, w_h, z_h, out_h,
             acc, idx_s, idx_r, ps, pr, o_r, o_s, wv):
        c = lax.axis_index("c")
        s = lax.axis_index("s")
        wid = c * NS + s

        sync_copy = plm_helpers.sync_copy
        sync_copy(z_h.at[pl.ds(s * AS, AS)], acc.at[pl.ds(s * AS, AS)])
        sync_copy(w_h, wv)

        iota16 = lax.iota(jnp.int32, 16)
        col = [jnp.full((16,), k, jnp.int32) for k in range(16)]
        zero16 = jnp.zeros((16,), jnp.float32)

        def zcols(j, _):
            rows = iota16 + j * 16
            plsc.store_scatter(o_s, [rows, col[0]], zero16)
            for k in range(10, 16):
                plsc.store_scatter(o_r, [rows, col[k]], zero16)
                plsc.store_scatter(o_s, [rows, col[k]], zero16)
            return 0
        lax.fori_loop(0, CHUNK // 16, zcols, 0)

        plsc.subcore_barrier()

        wxv = wv[0]
        wyv = wv[1]
        wzv = wv[2]

        def chunk_body(t, _):
            base = wid * EW + t * CHUNK
            sync_copy(send_h.at[pl.ds(base, CHUNK)], idx_s)
            sync_copy(recv_h.at[pl.ds(base, CHUNK)], idx_r)
            sync_copy(pos_h.at[idx_s], ps)
            sync_copy(pos_h.at[idx_r], pr)

            def grp(j, _):
                rows = iota16 + j * 16
                psx = plsc.load_gather(ps, [rows, col[0]])
                psy = plsc.load_gather(ps, [rows, col[1]])
                psz = plsc.load_gather(ps, [rows, col[2]])
                prx = plsc.load_gather(pr, [rows, col[0]])
                pry = plsc.load_gather(pr, [rows, col[1]])
                prz = plsc.load_gather(pr, [rows, col[2]])
                vx = prx - psx
                vy = pry - psy
                vz = prz - psz
                sx2 = vx * vx
                sy2 = vy * vy
                sz2 = vz * vz
                r2 = sx2 * wxv + sy2 * wyv + sz2 * wzv
                inv = 1.0 / (1.0 + r2)
                e = (0.5 * r2) * inv
                f = inv * inv
                ux = f * vx
                uy = f * vy
                uz = f * vz
                p0 = ux * vx
                p1 = uy * vy
                p2 = uz * vz
                p3 = ux * vy
                p4 = ux * vz
                p5 = uy * vz
                plsc.store_scatter(o_r, [rows, col[0]], e)
                plsc.store_scatter(o_r, [rows, col[1]], ux)
                plsc.store_scatter(o_r, [rows, col[2]], uy)
                plsc.store_scatter(o_r, [rows, col[3]], uz)
                plsc.store_scatter(o_s, [rows, col[1]], -ux)
                plsc.store_scatter(o_s, [rows, col[2]], -uy)
                plsc.store_scatter(o_s, [rows, col[3]], -uz)
                for k, pv in ((0, p0), (1, p1), (2, p2),
                              (3, p3), (4, p4), (5, p5)):
                    plsc.store_scatter(o_r, [rows, col[_PCOL + k]], pv)
                    plsc.store_scatter(o_s, [rows, col[_PCOL + k]], pv)
                return 0

            lax.fori_loop(0, CHUNK // 16, grp, 0)

            sync_copy(o_r, acc.at[idx_r], add=True)
            sync_copy(o_s, acc.at[idx_s], add=True)
            return 0

        lax.fori_loop(0, NCH, chunk_body, 0)

        plsc.subcore_barrier()
        sync_copy(acc.at[pl.ds(s * AS, AS)],
                  out_h.at[c].at[pl.ds(s * AS, AS)])

    return kern(send, recv, pos8, w16, z16)


NPAD = 100352
NBLK = 784
RB = 8
GRID = NBLK // RB

_ORD9 = (0, 3, 4, 3, 1, 5, 4, 5, 2)


def _tc_post(p1, batchp, wneg, s9, num_graphs):
    def body(p1_ref, b_ref, wn_ref, s9_ref,
             ne_ref, fo_ref, av_ref, vir_ref, te_ref):
        i = pl.program_id(0)
        p1b = p1_ref[...]
        a1 = p1b[0] + p1b[1]
        e = a1[0]
        ne_ref[...] = e
        wn = wn_ref[...][0:3, 0:1].reshape(3, 1, 1)
        fo_ref[...] = a1[1:4] * wn
        p6 = a1[_PCOL:_PCOL + 6]
        s9v = s9_ref[...][0:9, 0:1].reshape(9, 1, 1)
        p9 = jnp.stack([p6[k] for k in _ORD9])
        av = s9v * p9
        av_ref[...] = av

        @pl.when(i == 0)
        def _():
            vir_ref[...] = jnp.zeros((16, 128), jnp.float32)
            te_ref[...] = jnp.zeros((16, 128), jnp.float32)

        vir_ref[0:9] += jnp.sum(av, axis=1)
        b = b_ref[...]
        for g in range(num_graphs):
            te_ref[g:g + 1, :] += jnp.sum(
                jnp.where(b == g, e, 0.0), axis=0, keepdims=True)

    return pl.pallas_call(
        body,
        grid=(GRID,),
        in_specs=[
            pl.BlockSpec((2, 16, RB, 128), lambda i: (0, 0, i, 0)),
            pl.BlockSpec((RB, 128), lambda i: (i, 0)),
            pl.BlockSpec((8, 128), lambda i: (0, 0)),
            pl.BlockSpec((16, 128), lambda i: (0, 0)),
        ],
        out_specs=[
            pl.BlockSpec((RB, 128), lambda i: (i, 0)),
            pl.BlockSpec((3, RB, 128), lambda i: (0, i, 0)),
            pl.BlockSpec((9, RB, 128), lambda i: (0, i, 0)),
            pl.BlockSpec((16, 128), lambda i: (0, 0)),
            pl.BlockSpec((16, 128), lambda i: (0, 0)),
        ],
        out_shape=(
            jax.ShapeDtypeStruct((NBLK, 128), jnp.float32),
            jax.ShapeDtypeStruct((3, NBLK, 128), jnp.float32),
            jax.ShapeDtypeStruct((9, NBLK, 128), jnp.float32),
            jax.ShapeDtypeStruct((16, 128), jnp.float32),
            jax.ShapeDtypeStruct((16, 128), jnp.float32),
        ),
    )(p1, batchp, wneg, s9)


def kernel(positions, edge_index, batch, local_or_ghost, cell, ptr, pair_weight):
    num_graphs = ptr.shape[0] - 1
    n = positions.shape[0]
    w = pair_weight.astype(jnp.float32)

    e_in = edge_index.shape[1]
    send = jnp.pad(edge_index[0].astype(jnp.int32), (0, EP - e_in))
    recv = jnp.pad(edge_index[1].astype(jnp.int32), (0, EP - e_in))
    pos8 = jnp.pad(positions.astype(jnp.float32), ((0, 0), (0, 5)))
    w16 = jnp.broadcast_to(w[:, None], (3, 16))
    z16 = jnp.zeros((NA, 16), jnp.float32)

    out = _sc_accumulate(send, recv, pos8, w16, z16)

    p1 = jnp.pad(out.transpose(0, 2, 1), ((0, 0), (0, 0), (0, NPAD - NA)))
    p1 = p1.reshape(2, 16, NBLK, 128)
    batchp = jnp.pad(batch.astype(jnp.int32), (0, NPAD - n)).reshape(NBLK, 128)
    wneg = jnp.broadcast_to((-w)[:, None], (3, 128))
    wneg = jnp.pad(wneg, ((0, 5), (0, 0)))
    s9 = ((w[:, None] + w[None, :]) / 4.0).reshape(9)
    s9 = jnp.pad(jnp.broadcast_to(s9[:, None], (9, 128)), ((0, 7), (0, 0)))

    ne_b, fo_b, av_b, vir_b, te_b = _tc_post(p1, batchp, wneg, s9, num_graphs)

    node_energy = ne_b.reshape(-1)[:n]
    forces = fo_b.reshape(3, -1)[:, :n].transpose(1, 0)
    atom_virial = av_b.reshape(9, -1)[:, :n].transpose(1, 0).reshape(n, 3, 3)
    virials = jnp.sum(vir_b[0:9], axis=1).reshape(3, 3)
    total_energy_local = jnp.sum(te_b[0:num_graphs], axis=1)

    return (total_energy_local, node_energy, forces, virials, atom_virial)
